# Initial kernel scaffold; baseline (speedup 1.0000x reference)
import jax
import jax.numpy as jnp
from jax.experimental import pallas as pl


def kernel(x, edge_index, batch, params):
    raise NotImplementedError("write your pallas kernel here")


# speedup vs baseline: 1.0008x; 1.0008x over previous
import functools

import jax
import jax.numpy as jnp
from jax.experimental import pallas as pl


def _layer_norm(x, g, b, eps=1e-5):
    m = jnp.mean(x, -1, keepdims=True)
    v = jnp.var(x, -1, keepdims=True)
    return (x - m) / jnp.sqrt(v + eps) * g + b


def _fga_conv(x, src, dst, N, p):
    h = x @ p["W"]
    e = jax.nn.leaky_relu((h @ p["a_s"])[src] + (h @ p["a_d"])[dst], 0.2)
    emax = jax.ops.segment_max(e, dst, num_segments=N)
    emax = jnp.where(jnp.isfinite(emax), emax, 0.0)
    ex = jnp.exp(e - emax[dst])
    denom = jax.ops.segment_sum(ex, dst, num_segments=N)
    alpha = ex / jnp.maximum(denom[dst], 1e-16)
    agg = jax.ops.segment_sum(alpha[:, None] * h[src], dst, num_segments=N)
    gate = jax.nn.sigmoid(jnp.concatenate([h, agg], -1) @ p["Wg"] + p["bg"])
    return gate * (h + agg) + (1.0 - gate) * (h - agg)


def _semantic(x, p):
    s = jnp.tanh(x @ p["A"] + p["ab"]) @ p["B"]
    return x * jax.nn.sigmoid(s)


def _head_body(*refs):
    feats_ref, shell_mask_ref, *w_refs, o_ref = refs
    (pln_g, pln_b, pW, pb,
     b0_ln_g, b0_ln_b, b0_Win, b0_bin, b0_a, b0_Wg, b0_bg, b0_Wout, b0_bout,
     b1_ln_g, b1_ln_b, b1_Win, b1_bin, b1_a, b1_Wg, b1_bg, b1_Wout, b1_bout,
     ro_g, ro_b, W1, b1, W2, b2) = [r[...] for r in w_refs]
    feats = feats_ref[...]
    mask = shell_mask_ref[...]

    h = jax.nn.relu(_layer_norm(feats, pln_g, pln_b) @ pW + pb)
    r = h

    def radial(xs, ln_g, ln_b, Win, bin_, a, Wg, bg, Wout, bout):
        xn = _layer_norm(xs, ln_g, ln_b) * mask[..., None]
        G, T, D = xn.shape
        u = jax.nn.relu(xn.reshape(G * T, D) @ Win + bin_).reshape(G, T, D)
        decay = jax.nn.sigmoid(a)
        s = jnp.zeros_like(u[:, 0])
        outs = []
        for t in range(T):
            s = decay * s + u[:, t]
            outs.append(s)
        hh = jnp.stack(outs, axis=1)
        gate = jax.nn.sigmoid(xn.reshape(G * T, D) @ Wg + bg).reshape(G, T, D)
        return ((hh * gate).reshape(G * T, D) @ Wout + bout).reshape(G, T, D) * mask[..., None]

    r = r + radial(r, b0_ln_g, b0_ln_b, b0_Win, b0_bin, b0_a, b0_Wg, b0_bg, b0_Wout, b0_bout)
    r = r + radial(r, b1_ln_g, b1_ln_b, b1_Win, b1_bin, b1_a, b1_Wg, b1_bg, b1_Wout, b1_bout)
    g = _layer_norm(r[:, 0], ro_g, ro_b)
    g = jax.nn.relu(g @ W1 + b1)
    g = g @ W2 + b2
    o_ref[...] = g[:, 0]


def kernel(x, edge_index, batch, params):
    num_graphs = 64
    N = x.shape[0]
    src, dst = edge_index[0], edge_index[1]
    max_hop = 2
    num_shells = max_hop + 1
    labels = jnp.argmax(x, -1)
    shell = jnp.minimum(labels // 2, max_hop)
    side = labels % 2
    x1 = _semantic(_fga_conv(x, src, dst, N, params["conv1"]), params["sem1"])
    x2 = _semantic(_fga_conv(x1, src, dst, N, params["conv2"]), params["sem2"])
    x3 = _semantic(_fga_conv(x2, src, dst, N, params["conv3"]), params["sem3"])
    ns = jnp.concatenate([x1, x2, x3], -1)
    D = ns.shape[-1]
    flat = batch * (num_shells * 2) + shell * 2 + side
    slots = num_graphs * num_shells * 2
    psum = jax.ops.segment_sum(ns, flat, num_segments=slots)
    counts = jax.ops.segment_sum(jnp.ones((N,), ns.dtype), flat, num_segments=slots)
    pmax = jax.ops.segment_max(ns, flat, num_segments=slots)
    slot_mask = counts > 0
    pmean = psum / jnp.maximum(counts, 1.0)[:, None]
    pmax = jnp.where(slot_mask[:, None], pmax, 0.0)
    pmean = pmean.reshape(num_graphs, num_shells, 2, D)
    pmax = pmax.reshape(num_graphs, num_shells, 2, D)
    counts = counts.reshape(num_graphs, num_shells, 2)
    shell_mask = (counts.sum(-1) > 0).astype(ns.dtype)
    feats = jnp.concatenate(
        [pmean[:, :, 0], pmax[:, :, 0], pmean[:, :, 1], pmax[:, :, 1], jnp.log1p(counts)], -1)

    p = params["proj"]
    b0, b1b = params["block0"], params["block1"]
    weights = [
        p["ln_g"], p["ln_b"], p["W"], p["b"],
        b0["ln_g"], b0["ln_b"], b0["Win"], b0["bin"], b0["a"], b0["Wg"], b0["bg"], b0["Wout"], b0["bout"],
        b1b["ln_g"], b1b["ln_b"], b1b["Win"], b1b["bin"], b1b["a"], b1b["Wg"], b1b["bg"], b1b["Wout"], b1b["bout"],
        params["ro_g"], params["ro_b"], params["W1"], params["b1"], params["W2"], params["b2"],
    ]
    out = pl.pallas_call(
        _head_body,
        out_shape=jax.ShapeDtypeStruct((num_graphs,), jnp.float32),
    )(feats, shell_mask, *weights)
    return out


# speedup vs baseline: 14.8185x; 14.8066x over previous
import functools

import jax
import jax.numpy as jnp
from jax import lax
from jax.experimental import pallas as pl
from jax.experimental.pallas import tpu as pltpu
from jax.experimental.pallas import tpu_sc as plsc

N_NODES = 10000
N_EDGES = 320000
NW = 32
EPW = N_EDGES // NW
K = 80
NCHUNK = EPW // K
NP = 10240
RPT = NP // 16


def _leaky(z):
    return jnp.where(z >= 0, z, 0.2 * z)


def _make_sc_conv(CW):
    mesh = plsc.VectorSubcoreMesh(core_axis_name="c", subcore_axis_name="s")

    def body(h_hbm, es_hbm, ed_hbm, g_hbm, src_hbm, dst_hbm, zrows_hbm, zden_hbm,
             acc_hbm, den_hbm,
             es_v, ed_v, gv, src_v, dst_v, ex_v, rows_v, den_v, acc_sh, den_sh, sem):
        ci = lax.axis_index("c")
        si = lax.axis_index("s")
        wid = ci * 16 + si

        pltpu.sync_copy(es_hbm, es_v)
        pltpu.sync_copy(ed_hbm, ed_v)
        pltpu.sync_copy(g_hbm, gv)

        pltpu.sync_copy(zrows_hbm, acc_sh.at[pl.ds(si * RPT, RPT)])

        def zden_body(i, c):
            den_v[pl.ds(i * 16, 16)] = jnp.zeros((16,), jnp.float32)
            return c

        lax.fori_loop(0, RPT // 16, zden_body, 0)
        pltpu.sync_copy(den_v, den_sh.at[pl.ds(si * RPT, RPT)])
        plsc.subcore_barrier()

        gm = gv[...]
        ebase = wid * EPW

        def chunk_body(i, carry):
            base = ebase + i * K
            pltpu.sync_copy(src_hbm.at[pl.ds(base, K)], src_v)
            pltpu.sync_copy(dst_hbm.at[pl.ds(base, K)], dst_v)
            for j in range(K // 16):
                sl = pl.ds(j * 16, 16)
                sidx = src_v[sl]
                didx = dst_v[sl]
                z = plsc.load_gather(es_v, [sidx]) + plsc.load_gather(ed_v, [didx])
                ex_v[sl] = jnp.exp(_leaky(z) - gm)
            pltpu.sync_copy(ex_v, den_sh.at[dst_v], add=True)
            pltpu.async_copy(h_hbm.at[src_v], rows_v, sem).wait()

            def scale_body(j, c):
                bc = plsc.load_gather(ex_v, [jnp.full((16,), j, jnp.int32)])
                rr = rows_v.at[j]
                for cg in range(CW // 16):
                    cs = pl.ds(cg * 16, 16)
                    rr[cs] = rr[cs] * bc
                return c

            lax.fori_loop(0, K, scale_body, 0)
            pltpu.sync_copy(rows_v, acc_sh.at[dst_v], add=True)
            return carry

        lax.fori_loop(0, NCHUNK, chunk_body, 0)
        plsc.subcore_barrier()

        rsl = pl.ds(si * RPT, RPT)
        pltpu.sync_copy(acc_sh.at[rsl], acc_hbm.at[ci, rsl])
        pltpu.sync_copy(den_sh.at[rsl], den_v)
        pltpu.sync_copy(den_v, den_hbm.at[pl.ds(ci * NP + si * RPT, RPT)])

    return pl.kernel(
        body,
        out_type=(
            jax.ShapeDtypeStruct((2, NP, CW), jnp.float32),
            jax.ShapeDtypeStruct((2 * NP,), jnp.float32),
        ),
        mesh=mesh,
        compiler_params=pltpu.CompilerParams(needs_layout_passes=False),
        scratch_types=[
            pltpu.VMEM((N_NODES,), jnp.float32),
            pltpu.VMEM((N_NODES,), jnp.float32),
            pltpu.VMEM((16,), jnp.float32),
            pltpu.VMEM((K,), jnp.int32),
            pltpu.VMEM((K,), jnp.int32),
            pltpu.VMEM((K,), jnp.float32),
            pltpu.VMEM((K, CW), jnp.float32),
            pltpu.VMEM((RPT,), jnp.float32),
            pltpu.VMEM_SHARED((NP, CW), jnp.float32),
            pltpu.VMEM_SHARED((NP,), jnp.float32),
            pltpu.SemaphoreType.DMA,
        ],
    )


_sc_conv = {cw: _make_sc_conv(cw) for cw in (128,)}


def _fga_conv_sc(x, src, dst, p):
    N = x.shape[0]
    h = x @ p["W"]
    es = h @ p["a_s"]
    ed = h @ p["a_d"]
    g = _leaky(jnp.max(es) + jnp.max(ed))
    gvec = jnp.full((16,), g, jnp.float32)
    dout = h.shape[1]
    CW = 128
    hg = jnp.concatenate([h, jnp.zeros((N, CW - dout), jnp.float32)], -1) if dout < CW else h
    zrows = jnp.zeros((RPT, CW), jnp.float32)
    zden = jnp.zeros((RPT,), jnp.float32)
    agg_chunks = []
    den = None
    for c in range(hg.shape[1] // CW):
        hc = hg[:, c * CW:(c + 1) * CW]
        acc2, den2 = _sc_conv[CW](hc, es, ed, gvec, src, dst, zrows, zden)
        agg_chunks.append((acc2[0] + acc2[1])[:N])
        if den is None:
            den = (den2[:NP] + den2[NP:])[:N]
    agg_raw = jnp.concatenate(agg_chunks, axis=-1) if len(agg_chunks) > 1 else agg_chunks[0]
    agg = agg_raw[:, :dout] / jnp.maximum(den, 1e-16)[:, None]
    gate = jax.nn.sigmoid(jnp.concatenate([h, agg], -1) @ p["Wg"] + p["bg"])
    return gate * (h + agg) + (1.0 - gate) * (h - agg)


def _layer_norm(x, g, b, eps=1e-5):
    m = jnp.mean(x, -1, keepdims=True)
    v = jnp.var(x, -1, keepdims=True)
    return (x - m) / jnp.sqrt(v + eps) * g + b


def _semantic(x, p):
    s = jnp.tanh(x @ p["A"] + p["ab"]) @ p["B"]
    return x * jax.nn.sigmoid(s)


def _head_body(*refs):
    feats_ref, shell_mask_ref, *w_refs, o_ref = refs
    (pln_g, pln_b, pW, pb,
     b0_ln_g, b0_ln_b, b0_Win, b0_bin, b0_a, b0_Wg, b0_bg, b0_Wout, b0_bout,
     b1_ln_g, b1_ln_b, b1_Win, b1_bin, b1_a, b1_Wg, b1_bg, b1_Wout, b1_bout,
     ro_g, ro_b, W1, b1, W2, b2) = [r[...] for r in w_refs]
    feats = feats_ref[...]
    mask = shell_mask_ref[...]

    h = jax.nn.relu(_layer_norm(feats, pln_g, pln_b) @ pW + pb)
    r = h

    def radial(xs, ln_g, ln_b, Win, bin_, a, Wg, bg, Wout, bout):
        xn = _layer_norm(xs, ln_g, ln_b) * mask[..., None]
        G, T, D = xn.shape
        u = jax.nn.relu(xn.reshape(G * T, D) @ Win + bin_).reshape(G, T, D)
        decay = jax.nn.sigmoid(a)
        s = jnp.zeros_like(u[:, 0])
        outs = []
        for t in range(T):
            s = decay * s + u[:, t]
            outs.append(s)
        hh = jnp.stack(outs, axis=1)
        gate = jax.nn.sigmoid(xn.reshape(G * T, D) @ Wg + bg).reshape(G, T, D)
        return ((hh * gate).reshape(G * T, D) @ Wout + bout).reshape(G, T, D) * mask[..., None]

    r = r + radial(r, b0_ln_g, b0_ln_b, b0_Win, b0_bin, b0_a, b0_Wg, b0_bg, b0_Wout, b0_bout)
    r = r + radial(r, b1_ln_g, b1_ln_b, b1_Win, b1_bin, b1_a, b1_Wg, b1_bg, b1_Wout, b1_bout)
    g = _layer_norm(r[:, 0], ro_g, ro_b)
    g = jax.nn.relu(g @ W1 + b1)
    g = g @ W2 + b2
    o_ref[...] = g[:, 0]


def kernel(x, edge_index, batch, params):
    num_graphs = 64
    N = x.shape[0]
    src, dst = edge_index[0], edge_index[1]
    max_hop = 2
    num_shells = max_hop + 1
    labels = jnp.argmax(x, -1)
    shell = jnp.minimum(labels // 2, max_hop)
    side = labels % 2
    x1 = _semantic(_fga_conv_sc(x, src, dst, params["conv1"]), params["sem1"])
    x2 = _semantic(_fga_conv_sc(x1, src, dst, params["conv2"]), params["sem2"])
    x3 = _semantic(_fga_conv_sc(x2, src, dst, params["conv3"]), params["sem3"])
    ns = jnp.concatenate([x1, x2, x3], -1)
    D = ns.shape[-1]
    flat = batch * (num_shells * 2) + shell * 2 + side
    slots = num_graphs * num_shells * 2
    psum = jax.ops.segment_sum(ns, flat, num_segments=slots)
    counts = jax.ops.segment_sum(jnp.ones((N,), ns.dtype), flat, num_segments=slots)
    pmax = jax.ops.segment_max(ns, flat, num_segments=slots)
    slot_mask = counts > 0
    pmean = psum / jnp.maximum(counts, 1.0)[:, None]
    pmax = jnp.where(slot_mask[:, None], pmax, 0.0)
    pmean = pmean.reshape(num_graphs, num_shells, 2, D)
    pmax = pmax.reshape(num_graphs, num_shells, 2, D)
    counts = counts.reshape(num_graphs, num_shells, 2)
    shell_mask = (counts.sum(-1) > 0).astype(ns.dtype)
    feats = jnp.concatenate(
        [pmean[:, :, 0], pmax[:, :, 0], pmean[:, :, 1], pmax[:, :, 1], jnp.log1p(counts)], -1)

    p = params["proj"]
    b0, b1b = params["block0"], params["block1"]
    weights = [
        p["ln_g"], p["ln_b"], p["W"], p["b"],
        b0["ln_g"], b0["ln_b"], b0["Win"], b0["bin"], b0["a"], b0["Wg"], b0["bg"], b0["Wout"], b0["bout"],
        b1b["ln_g"], b1b["ln_b"], b1b["Win"], b1b["bin"], b1b["a"], b1b["Wg"], b1b["bg"], b1b["Wout"], b1b["bout"],
        params["ro_g"], params["ro_b"], params["W1"], params["b1"], params["W2"], params["b2"],
    ]
    out = pl.pallas_call(
        _head_body,
        out_shape=jax.ShapeDtypeStruct((num_graphs,), jnp.float32),
    )(feats, shell_mask, *weights)
    return out


# speedup vs baseline: 24.0901x; 1.6257x over previous
import functools

import jax
import jax.numpy as jnp
from jax import lax
from jax.experimental import pallas as pl
from jax.experimental.pallas import tpu as pltpu
from jax.experimental.pallas import tpu_sc as plsc

N_NODES = 10000
N_EDGES = 320000
NW = 32
EPW = N_EDGES // NW
K = 80
NCHUNK = EPW // K
NP = 10240
RPT = NP // 16


def _leaky(z):
    return jnp.where(z >= 0, z, 0.2 * z)


def _make_sc_conv(CW):
    mesh = plsc.VectorSubcoreMesh(core_axis_name="c", subcore_axis_name="s")

    def body(h_hbm, es_hbm, ed_hbm, g_hbm, src_hbm, dst_hbm, zrows_hbm,
             acc_hbm, den_hbm,
             es_v, ed_v, gv,
             src_b, dst_b, dsc_b, ex_b, rows_b, den_v, acc_sh, den_sh,
             sem_idx, sem_rows, sem_den, sem_sc):
        ci = lax.axis_index("c")
        si = lax.axis_index("s")
        wid = ci * 16 + si

        pltpu.sync_copy(es_hbm, es_v)
        pltpu.sync_copy(ed_hbm, ed_v)
        pltpu.sync_copy(g_hbm, gv)

        pltpu.sync_copy(zrows_hbm, acc_sh.at[pl.ds(si * RPT, RPT)])

        def zden_body(i, c):
            den_v[pl.ds(i * 16, 16)] = jnp.zeros((16,), jnp.float32)
            return c

        lax.fori_loop(0, RPT // 16, zden_body, 0)
        pltpu.sync_copy(den_v, den_sh.at[pl.ds(si * RPT, RPT)])
        plsc.subcore_barrier()

        gm = gv[...]
        ebase = wid * EPW

        def issue_idx(i, b):
            base = ebase + i * K
            pltpu.async_copy(src_hbm.at[pl.ds(base, K)], src_b[b], sem_idx[b])
            pltpu.async_copy(dst_hbm.at[pl.ds(base, K)], dst_b[b], sem_idx[b])

        def wait_idx(b):
            pltpu.make_async_copy(src_hbm.at[pl.ds(0, K)], src_b[b], sem_idx[b]).wait()
            pltpu.make_async_copy(dst_hbm.at[pl.ds(0, K)], dst_b[b], sem_idx[b]).wait()

        def wait_scatters(b):
            pltpu.make_async_copy(ex_b[b], den_sh.at[dsc_b[b]], sem_den[b]).wait()
            pltpu.make_async_copy(rows_b[b], acc_sh.at[dsc_b[b]], sem_sc[b]).wait()

        def sub_iter(i, b, nb, guard, prefetch):
            if prefetch:
                issue_idx(i + 1, nb)
            wait_idx(b)
            if guard is None:
                wait_scatters(b)
            else:
                @pl.when(guard)
                def _():
                    wait_scatters(b)
            pltpu.async_copy(h_hbm.at[src_b[b]], rows_b[b], sem_rows[b])
            for j in range(K // 16):
                sl = pl.ds(j * 16, 16)
                z = plsc.load_gather(es_v, [src_b[b][sl]]) \
                    + plsc.load_gather(ed_v, [dst_b[b][sl]])
                ex_b[b][sl] = jnp.exp(_leaky(z) - gm)
                dsc_b[b][sl] = dst_b[b][sl]
            pltpu.async_copy(ex_b[b], den_sh.at[dsc_b[b]], sem_den[b], add=True)
            pltpu.make_async_copy(h_hbm.at[pl.ds(0, K)], rows_b[b], sem_rows[b]).wait()

            @plsc.parallel_loop(0, K, 1, unroll=4)
            def _scale(j):
                bc = plsc.load_gather(ex_b[b], [jnp.full((16,), j, jnp.int32)])
                rr = rows_b[b].at[j]
                for cg in range(CW // 16):
                    cs = pl.ds(cg * 16, 16)
                    rr[cs] = rr[cs] * bc

            pltpu.async_copy(rows_b[b], acc_sh.at[dsc_b[b]], sem_sc[b], add=True)

        issue_idx(0, 0)

        def pair_body(k, c):
            g = k > 0
            sub_iter(2 * k, 0, 1, g, True)
            sub_iter(2 * k + 1, 1, 0, g, True)
            return c

        lax.fori_loop(0, (NCHUNK - 1) // 2, pair_body, 0)
        sub_iter(NCHUNK - 1, 0, 1, None, False)
        wait_scatters(1)
        wait_scatters(0)
        plsc.subcore_barrier()

        rsl = pl.ds(si * RPT, RPT)
        pltpu.sync_copy(acc_sh.at[rsl], acc_hbm.at[ci, rsl])
        pltpu.sync_copy(den_sh.at[rsl], den_v)
        pltpu.sync_copy(den_v, den_hbm.at[pl.ds(ci * NP + si * RPT, RPT)])

    return pl.kernel(
        body,
        out_type=(
            jax.ShapeDtypeStruct((2, NP, CW), jnp.float32),
            jax.ShapeDtypeStruct((2 * NP,), jnp.float32),
        ),
        mesh=mesh,
        compiler_params=pltpu.CompilerParams(needs_layout_passes=False),
        scratch_types=[
            pltpu.VMEM((N_NODES,), jnp.float32),
            pltpu.VMEM((N_NODES,), jnp.float32),
            pltpu.VMEM((16,), jnp.float32),
            (pltpu.VMEM((K,), jnp.int32),) * 2,
            (pltpu.VMEM((K,), jnp.int32),) * 2,
            (pltpu.VMEM((K,), jnp.int32),) * 2,
            (pltpu.VMEM((K,), jnp.float32),) * 2,
            (pltpu.VMEM((K, CW), jnp.float32),) * 2,
            pltpu.VMEM((RPT,), jnp.float32),
            pltpu.VMEM_SHARED((NP, CW), jnp.float32),
            pltpu.VMEM_SHARED((NP,), jnp.float32),
            (pltpu.SemaphoreType.DMA,) * 2,
            (pltpu.SemaphoreType.DMA,) * 2,
            (pltpu.SemaphoreType.DMA,) * 2,
            (pltpu.SemaphoreType.DMA,) * 2,
        ],
    )


_sc_conv = {cw: _make_sc_conv(cw) for cw in (128,)}


def _fga_conv_sc(x, src, dst, p):
    N = x.shape[0]
    h = x @ p["W"]
    es = h @ p["a_s"]
    ed = h @ p["a_d"]
    g = _leaky(jnp.max(es) + jnp.max(ed))
    gvec = jnp.full((16,), g, jnp.float32)
    dout = h.shape[1]
    CW = 128
    hg = jnp.concatenate([h, jnp.zeros((N, CW - dout), jnp.float32)], -1) if dout < CW else h
    zrows = jnp.zeros((RPT, CW), jnp.float32)
    agg_chunks = []
    den = None
    for c in range(hg.shape[1] // CW):
        hc = hg[:, c * CW:(c + 1) * CW]
        acc2, den2 = _sc_conv[CW](hc, es, ed, gvec, src, dst, zrows)
        agg_chunks.append((acc2[0] + acc2[1])[:N])
        if den is None:
            den = (den2[:NP] + den2[NP:])[:N]
    agg_raw = jnp.concatenate(agg_chunks, axis=-1) if len(agg_chunks) > 1 else agg_chunks[0]
    agg = agg_raw[:, :dout] / jnp.maximum(den, 1e-16)[:, None]
    gate = jax.nn.sigmoid(jnp.concatenate([h, agg], -1) @ p["Wg"] + p["bg"])
    return gate * (h + agg) + (1.0 - gate) * (h - agg)


def _layer_norm(x, g, b, eps=1e-5):
    m = jnp.mean(x, -1, keepdims=True)
    v = jnp.var(x, -1, keepdims=True)
    return (x - m) / jnp.sqrt(v + eps) * g + b


def _semantic(x, p):
    s = jnp.tanh(x @ p["A"] + p["ab"]) @ p["B"]
    return x * jax.nn.sigmoid(s)


def _head_body(*refs):
    feats_ref, shell_mask_ref, *w_refs, o_ref = refs
    (pln_g, pln_b, pW, pb,
     b0_ln_g, b0_ln_b, b0_Win, b0_bin, b0_a, b0_Wg, b0_bg, b0_Wout, b0_bout,
     b1_ln_g, b1_ln_b, b1_Win, b1_bin, b1_a, b1_Wg, b1_bg, b1_Wout, b1_bout,
     ro_g, ro_b, W1, b1, W2, b2) = [r[...] for r in w_refs]
    feats = feats_ref[...]
    mask = shell_mask_ref[...]

    h = jax.nn.relu(_layer_norm(feats, pln_g, pln_b) @ pW + pb)
    r = h

    def radial(xs, ln_g, ln_b, Win, bin_, a, Wg, bg, Wout, bout):
        xn = _layer_norm(xs, ln_g, ln_b) * mask[..., None]
        G, T, D = xn.shape
        u = jax.nn.relu(xn.reshape(G * T, D) @ Win + bin_).reshape(G, T, D)
        decay = jax.nn.sigmoid(a)
        s = jnp.zeros_like(u[:, 0])
        outs = []
        for t in range(T):
            s = decay * s + u[:, t]
            outs.append(s)
        hh = jnp.stack(outs, axis=1)
        gate = jax.nn.sigmoid(xn.reshape(G * T, D) @ Wg + bg).reshape(G, T, D)
        return ((hh * gate).reshape(G * T, D) @ Wout + bout).reshape(G, T, D) * mask[..., None]

    r = r + radial(r, b0_ln_g, b0_ln_b, b0_Win, b0_bin, b0_a, b0_Wg, b0_bg, b0_Wout, b0_bout)
    r = r + radial(r, b1_ln_g, b1_ln_b, b1_Win, b1_bin, b1_a, b1_Wg, b1_bg, b1_Wout, b1_bout)
    g = _layer_norm(r[:, 0], ro_g, ro_b)
    g = jax.nn.relu(g @ W1 + b1)
    g = g @ W2 + b2
    o_ref[...] = g[:, 0]


def kernel(x, edge_index, batch, params):
    num_graphs = 64
    N = x.shape[0]
    src, dst = edge_index[0], edge_index[1]
    max_hop = 2
    num_shells = max_hop + 1
    labels = jnp.argmax(x, -1)
    shell = jnp.minimum(labels // 2, max_hop)
    side = labels % 2
    x1 = _semantic(_fga_conv_sc(x, src, dst, params["conv1"]), params["sem1"])
    x2 = _semantic(_fga_conv_sc(x1, src, dst, params["conv2"]), params["sem2"])
    x3 = _semantic(_fga_conv_sc(x2, src, dst, params["conv3"]), params["sem3"])
    ns = jnp.concatenate([x1, x2, x3], -1)
    D = ns.shape[-1]
    flat = batch * (num_shells * 2) + shell * 2 + side
    slots = num_graphs * num_shells * 2
    psum = jax.ops.segment_sum(ns, flat, num_segments=slots)
    counts = jax.ops.segment_sum(jnp.ones((N,), ns.dtype), flat, num_segments=slots)
    pmax = jax.ops.segment_max(ns, flat, num_segments=slots)
    slot_mask = counts > 0
    pmean = psum / jnp.maximum(counts, 1.0)[:, None]
    pmax = jnp.where(slot_mask[:, None], pmax, 0.0)
    pmean = pmean.reshape(num_graphs, num_shells, 2, D)
    pmax = pmax.reshape(num_graphs, num_shells, 2, D)
    counts = counts.reshape(num_graphs, num_shells, 2)
    shell_mask = (counts.sum(-1) > 0).astype(ns.dtype)
    feats = jnp.concatenate(
        [pmean[:, :, 0], pmax[:, :, 0], pmean[:, :, 1], pmax[:, :, 1], jnp.log1p(counts)], -1)

    p = params["proj"]
    b0, b1b = params["block0"], params["block1"]
    weights = [
        p["ln_g"], p["ln_b"], p["W"], p["b"],
        b0["ln_g"], b0["ln_b"], b0["Win"], b0["bin"], b0["a"], b0["Wg"], b0["bg"], b0["Wout"], b0["bout"],
        b1b["ln_g"], b1b["ln_b"], b1b["Win"], b1b["bin"], b1b["a"], b1b["Wg"], b1b["bg"], b1b["Wout"], b1b["bout"],
        params["ro_g"], params["ro_b"], params["W1"], params["b1"], params["W2"], params["b2"],
    ]
    out = pl.pallas_call(
        _head_body,
        out_shape=jax.ShapeDtypeStruct((num_graphs,), jnp.float32),
    )(feats, shell_mask, *weights)
    return out
